# Initial kernel scaffold; baseline (speedup 1.0000x reference)
import jax
import jax.numpy as jnp
from jax.experimental import pallas as pl


def kernel(xyz, points, targets, w0, b0, g0, be0, w1, b1, g1, be1, w2, b2, g2, be2, att_a):
    raise NotImplementedError("write your pallas kernel here")


# speedup vs baseline: 1.5355x; 1.5355x over previous
import functools

import jax
import jax.numpy as jnp
from jax import lax
from jax.experimental import pallas as pl
from jax.experimental.pallas import tpu as pltpu

B, N, D_FEAT = 16, 4096, 6
NPOINT, RADIUS, NSAMPLE = 1024, 0.2, 32
MLP_DIMS = [32, 32, 64]
EPS = 1e-5
ALPHA = 0.2


def _fps_body(xs_ref, ys_ref, zs_ref, start_ref, out_ref):
    xs = xs_ref[...]
    ys = ys_ref[...]
    zs = zs_ref[...]
    col_iota = lax.broadcasted_iota(jnp.int32, (B, N), 1)
    lane_iota = lax.broadcasted_iota(jnp.int32, (B, NPOINT), 1)

    def step(t, carry):
        dist, far, out = carry
        out = jnp.where(lane_iota == t, far, out)
        oh = (col_iota == far).astype(jnp.float32)
        cx = jnp.sum(xs * oh, axis=1, keepdims=True)
        cy = jnp.sum(ys * oh, axis=1, keepdims=True)
        cz = jnp.sum(zs * oh, axis=1, keepdims=True)
        d = (xs - cx) ** 2 + (ys - cy) ** 2
        d = d + (zs - cz) ** 2
        dist = jnp.minimum(dist, d)
        m = jnp.max(dist, axis=1, keepdims=True)
        far = jnp.min(jnp.where(dist == m, col_iota, N), axis=1, keepdims=True)
        return dist, far, out

    dist0 = jnp.full((B, N), 1e10, jnp.float32)
    out0 = jnp.zeros((B, NPOINT), jnp.int32)
    _, _, out = lax.fori_loop(0, NPOINT, step, (dist0, start_ref[...], out0))
    out_ref[...] = out


def _run_fps(xyz, start):
    return pl.pallas_call(
        _fps_body,
        out_shape=jax.ShapeDtypeStruct((B, NPOINT), jnp.int32),
    )(xyz[:, 0, :], xyz[:, 1, :], xyz[:, 2, :], start.reshape(B, 1))


def _mlp_att_body(np_ref, fp_ref, wt_refs, b_refs, sc_refs, be_refs,
                  attp_ref, atth_ref, out_ref, *, rows):
    h = np_ref[0]
    g = fp_ref[0]
    for i in range(3):
        wt = wt_refs[i][...]
        bb = b_refs[i][...]
        sc = sc_refs[i][...]
        be = be_refs[i][...]
        h = jax.nn.relu((jnp.dot(h, wt, preferred_element_type=jnp.float32)
                         + bb) * sc + be)
        g = jax.nn.relu((jnp.dot(g, wt, preferred_element_type=jnp.float32)
                         + bb) * sc + be)
    delta_p = -np_ref[0][:, 0:3]
    e_p = jnp.dot(delta_p, attp_ref[...], preferred_element_type=jnp.float32)
    e_h = jnp.dot(h, atth_ref[...], preferred_element_type=jnp.float32)
    e_g = jnp.dot(g, atth_ref[...], preferred_element_type=jnp.float32)
    e = (e_p - e_h).reshape(rows, NSAMPLE, MLP_DIMS[2])
    e = e + e_g.reshape(rows, 1, MLP_DIMS[2])
    e = jnp.where(e >= 0, e, ALPHA * e)
    m = jnp.max(e, axis=1, keepdims=True)
    p = jnp.exp(e - m)
    att = p / jnp.sum(p, axis=1, keepdims=True)
    hn = h.reshape(rows, NSAMPLE, MLP_DIMS[2])
    out_ref[0] = jnp.sum(att * hn, axis=1)


def _run_mlp_att(np_feat, fp_feat, params):
    rows = 128
    grid = (B, NPOINT // rows)
    wts = [jnp.transpose(params['w%d' % i]) for i in range(3)]
    bs = [params['b%d' % i].reshape(1, -1) for i in range(3)]
    scs = [(params['g%d' % i] / jnp.sqrt(1.0 + EPS)).reshape(1, -1)
           for i in range(3)]
    bes = [params['be%d' % i].reshape(1, -1) for i in range(3)]
    att_p = params['att_a'][0:3]
    att_h = params['att_a'][3:]

    def fixed(shape):
        return pl.BlockSpec(shape, lambda b, r: (0,) * len(shape))

    in_specs = (
        [pl.BlockSpec((1, rows * NSAMPLE, 9), lambda b, r: (b, r, 0)),
         pl.BlockSpec((1, rows, 9), lambda b, r: (b, r, 0))]
        + [fixed(w.shape) for w in wts]
        + [fixed(x.shape) for x in bs]
        + [fixed(x.shape) for x in scs]
        + [fixed(x.shape) for x in bes]
        + [fixed(att_p.shape), fixed(att_h.shape)]
    )

    def body(np_r, fp_r, w0r, w1r, w2r, b0r, b1r, b2r, s0r, s1r, s2r,
             e0r, e1r, e2r, apr, ahr, out_r):
        _mlp_att_body(np_r, fp_r, (w0r, w1r, w2r), (b0r, b1r, b2r),
                      (s0r, s1r, s2r), (e0r, e1r, e2r), apr, ahr, out_r,
                      rows=rows)

    return pl.pallas_call(
        body,
        grid=grid,
        in_specs=in_specs,
        out_specs=pl.BlockSpec((1, rows, MLP_DIMS[2]), lambda b, r: (b, r, 0)),
        out_shape=jax.ShapeDtypeStruct((B, NPOINT, MLP_DIMS[2]), jnp.float32),
    )(np_feat, fp_feat, *wts, *bs, *scs, *bes, att_p, att_h)


def _index_points(points, idx):
    return jax.vmap(lambda p, i: p[i])(points, idx)


def _query_ball(xyz_t, new_xyz):
    gi = jnp.broadcast_to(jnp.arange(N, dtype=jnp.int32), (B, NPOINT, N))
    dist = -2.0 * jnp.matmul(new_xyz, jnp.swapaxes(xyz_t, 1, 2))
    dist = dist + jnp.sum(new_xyz ** 2, -1)[:, :, None]
    dist = dist + jnp.sum(xyz_t ** 2, -1)[:, None, :]
    gi = jnp.where(dist > RADIUS ** 2, N, gi)
    gi = jnp.sort(gi, axis=-1)[:, :, :NSAMPLE]
    first = jnp.broadcast_to(gi[:, :, :1], gi.shape)
    return jnp.where(gi == N, first, gi)


def kernel(xyz, points, targets, w0, b0, g0, be0, w1, b1, g1, be1,
           w2, b2, g2, be2, att_a):
    params = {'w0': w0, 'b0': b0, 'g0': g0, 'be0': be0,
              'w1': w1, 'b1': b1, 'g1': g1, 'be1': be1,
              'w2': w2, 'b2': b2, 'g2': g2, 'be2': be2, 'att_a': att_a}
    xyz_t = jnp.swapaxes(xyz, 1, 2)
    pts_t = jnp.swapaxes(points, 1, 2)
    start = jax.random.randint(jax.random.key(1), (B,), 0, N, dtype=jnp.int32)

    fps_idx = _run_fps(xyz, start)

    new_xyz = _index_points(xyz_t, fps_idx)
    new_targets = _index_points(targets, fps_idx)
    fp_pts = _index_points(pts_t, fps_idx)
    idx = _query_ball(xyz_t, new_xyz)
    grouped_xyz = _index_points(xyz_t, idx)
    grouped_pts = _index_points(pts_t, idx)
    np_feat = jnp.concatenate(
        [grouped_xyz - new_xyz[:, :, None, :], grouped_pts], axis=-1)
    np_feat = np_feat.reshape(B, NPOINT * NSAMPLE, 9)
    fp_feat = jnp.concatenate([new_xyz, fp_pts], axis=-1)

    pooled = _run_mlp_att(np_feat, fp_feat, params)

    return (jnp.swapaxes(new_xyz, 1, 2),
            jnp.swapaxes(pooled, 1, 2),
            new_targets)


# speedup vs baseline: 17.0434x; 11.0992x over previous
import numpy as np

import jax
import jax.numpy as jnp
from jax import lax
from jax.experimental import pallas as pl
from jax.experimental.pallas import tpu as pltpu
from jax.experimental.pallas import tpu_sc as plsc

B, N, D_FEAT = 16, 4096, 6
NPOINT, RADIUS, NSAMPLE = 1024, 0.2, 32
MLP_DIMS = [32, 32, 64]
EPS = 1e-5
ALPHA = 0.2
R2 = np.float32(RADIUS ** 2)

ROWS_W = NPOINT // 2
CH = 128
NPROW = NSAMPLE * 9
NWORD = N // 16


def _fps_body(xs_ref, ys_ref, zs_ref, start_ref, out_ref):
    xs = xs_ref[...]
    ys = ys_ref[...]
    zs = zs_ref[...]
    col_iota = lax.broadcasted_iota(jnp.int32, (B, N), 1)
    lane_iota = lax.broadcasted_iota(jnp.int32, (B, NPOINT), 1)

    def step(t, carry):
        dist, far, out = carry
        out = jnp.where(lane_iota == t, far, out)
        oh = (col_iota == far).astype(jnp.float32)
        cx = jnp.sum(xs * oh, axis=1, keepdims=True)
        cy = jnp.sum(ys * oh, axis=1, keepdims=True)
        cz = jnp.sum(zs * oh, axis=1, keepdims=True)
        d = (xs - cx) ** 2 + (ys - cy) ** 2
        d = d + (zs - cz) ** 2
        dist = jnp.minimum(dist, d)
        m = jnp.max(dist, axis=1, keepdims=True)
        far = jnp.min(jnp.where(dist == m, col_iota, N), axis=1, keepdims=True)
        return dist, far, out

    dist0 = jnp.full((B, N), 1e10, jnp.float32)
    out0 = jnp.zeros((B, NPOINT), jnp.int32)
    _, _, out = lax.fori_loop(0, NPOINT, step, (dist0, start_ref[...], out0))
    out_ref[...] = out


def _run_fps(xyz, start):
    return pl.pallas_call(
        _fps_body,
        out_shape=jax.ShapeDtypeStruct((B, NPOINT), jnp.int32),
    )(xyz[:, 0, :], xyz[:, 1, :], xyz[:, 2, :], start.reshape(B, 1))


def _sc_gather_body(xyz_hbm, pts_hbm, tgt_hbm, fidx_hbm,
                    nxyz_out, ntgt_out, fp_out,
                    xv, yv, zv, p0, p1, p2, p3, p4, p5,
                    tg, fidx, ntst, fpst, nxst):
    wid = lax.axis_index("s") * 2 + lax.axis_index("c")
    b = wid // 2
    h = wid % 2
    io = lax.iota(jnp.int32, 16)
    pv = (p0, p1, p2, p3, p4, p5)

    pltpu.sync_copy(xyz_hbm.at[pl.ds((b * 3 + 0) * N, N)], xv)
    pltpu.sync_copy(xyz_hbm.at[pl.ds((b * 3 + 1) * N, N)], yv)
    pltpu.sync_copy(xyz_hbm.at[pl.ds((b * 3 + 2) * N, N)], zv)
    for c in range(6):
        pltpu.sync_copy(pts_hbm.at[pl.ds((b * 6 + c) * N, N)], pv[c])
    pltpu.sync_copy(tgt_hbm.at[pl.ds(b * N, N)], tg)
    pltpu.sync_copy(fidx_hbm.at[pl.ds(b * NPOINT + h * ROWS_W, ROWS_W)], fidx)

    def fp_body(v, _):
        idx = fidx[pl.ds(v * 16, 16)]
        gx = plsc.load_gather(xv, [idx])
        gy = plsc.load_gather(yv, [idx])
        gz = plsc.load_gather(zv, [idx])
        nxst[pl.ds(v * 16, 16)] = gx
        nxst[pl.ds(ROWS_W + v * 16, 16)] = gy
        nxst[pl.ds(2 * ROWS_W + v * 16, 16)] = gz
        ntst[pl.ds(v * 16, 16)] = plsc.load_gather(tg, [idx])
        rbase = (io + v * 16) * 9
        plsc.store_scatter(fpst, [rbase], gx)
        plsc.store_scatter(fpst, [rbase + 1], gy)
        plsc.store_scatter(fpst, [rbase + 2], gz)
        for c in range(6):
            plsc.store_scatter(fpst, [rbase + 3 + c],
                               plsc.load_gather(pv[c], [idx]))
        return 0

    lax.fori_loop(0, ROWS_W // 16, fp_body, 0)

    for c in range(3):
        pltpu.sync_copy(
            nxst.at[pl.ds(c * ROWS_W, ROWS_W)],
            nxyz_out.at[pl.ds((b * 3 + c) * NPOINT + h * ROWS_W, ROWS_W)])
    pltpu.sync_copy(ntst, ntgt_out.at[pl.ds(b * NPOINT + h * ROWS_W, ROWS_W)])
    pltpu.sync_copy(fpst, fp_out.at[pl.ds((b * NPOINT + h * ROWS_W) * 9,
                                          ROWS_W * 9)])


def _run_sc_gather(xyz, points, targets, fps_idx):
    mesh = plsc.VectorSubcoreMesh(core_axis_name="c", subcore_axis_name="s")
    f32, i32 = jnp.float32, jnp.int32
    out_type = (
        jax.ShapeDtypeStruct((B * 3 * NPOINT,), f32),
        jax.ShapeDtypeStruct((B * NPOINT,), i32),
        jax.ShapeDtypeStruct((B * NPOINT * 9,), f32),
    )
    scratch = (
        [pltpu.VMEM((N,), f32) for _ in range(3)]
        + [pltpu.VMEM((N,), f32) for _ in range(6)]
        + [pltpu.VMEM((N,), i32),
           pltpu.VMEM((ROWS_W,), i32),
           pltpu.VMEM((ROWS_W,), i32),
           pltpu.VMEM((ROWS_W * 9,), f32),
           pltpu.VMEM((3 * ROWS_W,), f32)]
    )
    fn = pl.kernel(_sc_gather_body, out_type=out_type, mesh=mesh,
                   scratch_types=scratch,
                   compiler_params=pltpu.CompilerParams(
                       needs_layout_passes=False))
    return fn(xyz.reshape(-1), points.reshape(-1), targets.reshape(-1),
              fps_idx.reshape(-1))


def _sqmask_body(fp_ref, xyz_ref, pw_ref, out_ref):
    src = fp_ref[0][:, 0:3]
    xr = xyz_ref[0]
    mm = lax.dot_general(src, xr, (((1,), (0,)), ((), ())),
                         preferred_element_type=jnp.float32)
    s2 = src * src
    sn = (s2[:, 0:1] + s2[:, 1:2]) + s2[:, 2:3]
    x2 = xr * xr
    dn = (x2[0:1, :] + x2[1:2, :]) + x2[2:3, :]
    sq = -2.0 * mm
    sq = sq + sn
    sq = sq + dn
    inr = jnp.logical_not(sq > R2).astype(jnp.float32)
    out_ref[0] = jnp.dot(inr, pw_ref[...],
                         preferred_element_type=jnp.float32)


def _run_sqmask(fp_flat, xyz):
    ts = 256
    grid = (B, NPOINT // ts)
    fp_feat = fp_flat.reshape(B, NPOINT, 9)
    jr = np.arange(N)
    pw = np.where((jr[:, None] // 16) == np.arange(NWORD)[None, :],
                  (1 << (jr[:, None] % 16)).astype(np.float32), 0.0)
    pw = jnp.asarray(pw, jnp.float32)
    return pl.pallas_call(
        _sqmask_body,
        grid=grid,
        in_specs=[pl.BlockSpec((1, ts, 9), lambda b, r: (b, r, 0)),
                  pl.BlockSpec((1, 3, N), lambda b, r: (b, 0, 0)),
                  pl.BlockSpec((N, NWORD), lambda b, r: (0, 0))],
        out_specs=pl.BlockSpec((1, ts, NWORD), lambda b, r: (b, r, 0)),
        out_shape=jax.ShapeDtypeStruct((B, NPOINT, NWORD), jnp.float32),
    )(fp_feat, xyz, pw)


def _sc_select_body(xyz_hbm, pts_hbm, nxyz_hbm, hw_hbm, np_out,
                    xv, yv, zv, p0, p1, p2, p3, p4, p5,
                    nx, ny, nz, npst, selbuf, hwbuf, sems):
    wid = lax.axis_index("s") * 2 + lax.axis_index("c")
    b = wid // 2
    h = wid % 2
    io = lax.iota(jnp.int32, 16)
    pv = (p0, p1, p2, p3, p4, p5)
    rowbase = b * NPOINT + h * ROWS_W

    pltpu.sync_copy(xyz_hbm.at[pl.ds((b * 3 + 0) * N, N)], xv)
    pltpu.sync_copy(xyz_hbm.at[pl.ds((b * 3 + 1) * N, N)], yv)
    pltpu.sync_copy(xyz_hbm.at[pl.ds((b * 3 + 2) * N, N)], zv)
    for c in range(6):
        pltpu.sync_copy(pts_hbm.at[pl.ds((b * 6 + c) * N, N)], pv[c])
    pltpu.sync_copy(nxyz_hbm.at[pl.ds((b * 3 + 0) * NPOINT + h * ROWS_W,
                                      ROWS_W)], nx)
    pltpu.sync_copy(nxyz_hbm.at[pl.ds((b * 3 + 1) * NPOINT + h * ROWS_W,
                                      ROWS_W)], ny)
    pltpu.sync_copy(nxyz_hbm.at[pl.ds((b * 3 + 2) * NPOINT + h * ROWS_W,
                                      ROWS_W)], nz)

    pltpu.async_copy(hw_hbm.at[pl.ds(rowbase * NWORD, NWORD)],
                     hwbuf.at[pl.ds(0, NWORD)], sems.at[0])

    def row_body(r, _):
        par = lax.rem(r, 2)
        nxt = lax.rem(r + 1, 2)

        @pl.when(r + 1 < ROWS_W)
        def _prefetch():
            pltpu.async_copy(
                hw_hbm.at[pl.ds((rowbase + r + 1) * NWORD, NWORD)],
                hwbuf.at[pl.ds(nxt * NWORD, NWORD)], sems.at[nxt])

        pltpu.make_async_copy(
            hw_hbm.at[pl.ds((rowbase + r) * NWORD, NWORD)],
            hwbuf.at[pl.ds(par * NWORD, NWORD)], sems.at[par]).wait()

        hwoff = par * NWORD

        def sel_cond(c):
            pos, j = c
            return jnp.logical_and(pos < NSAMPLE, j < NWORD)

        def sel_body(c):
            pos, j = c
            wf = plsc.load_gather(
                hwbuf, [jnp.broadcast_to(hwoff + j, (16,)).astype(jnp.int32)])
            w = wf.astype(jnp.int32)
            m = ((w >> io) & 1) == 1
            plsc.store_compressed(selbuf.at[pl.ds(pos, 16)], io + j * 16,
                                  mask=m)
            return pos + jnp.sum(m.astype(jnp.int32)), j + 1

        pos, _ = lax.while_loop(sel_cond, sel_body,
                                (jnp.int32(0), jnp.int32(0)))
        s0 = selbuf[pl.ds(0, 16)]
        s1 = selbuf[pl.ds(16, 16)]
        first = jnp.broadcast_to(s0[0], (16,))
        idx0 = jnp.where(io < pos, s0, first)
        idx1 = jnp.where(io + 16 < pos, s1, first)

        rsplat = jnp.broadcast_to(r, (16,)).astype(jnp.int32)
        cx = plsc.load_gather(nx, [rsplat])
        cy = plsc.load_gather(ny, [rsplat])
        cz = plsc.load_gather(nz, [rsplat])
        ro = lax.rem(r, CH) * NPROW
        for half, idxh in ((0, idx0), (1, idx1)):
            sb = ro + (io + half * 16) * 9
            plsc.store_scatter(npst, [sb], plsc.load_gather(xv, [idxh]) - cx)
            plsc.store_scatter(npst, [sb + 1],
                               plsc.load_gather(yv, [idxh]) - cy)
            plsc.store_scatter(npst, [sb + 2],
                               plsc.load_gather(zv, [idxh]) - cz)
            for c in range(6):
                plsc.store_scatter(npst, [sb + 3 + c],
                                   plsc.load_gather(pv[c], [idxh]))

        @pl.when(lax.rem(r, CH) == CH - 1)
        def _flush():
            off = (rowbase + (r - (CH - 1))) * NPROW
            pltpu.sync_copy(npst, np_out.at[pl.ds(off, CH * NPROW)])

        return 0

    lax.fori_loop(0, ROWS_W, row_body, 0)


def _run_sc_select(xyz, points, nxyz_flat, hw):
    mesh = plsc.VectorSubcoreMesh(core_axis_name="c", subcore_axis_name="s")
    f32 = jnp.float32
    out_type = jax.ShapeDtypeStruct((B * NPOINT * NPROW,), f32)
    scratch = (
        [pltpu.VMEM((N,), f32) for _ in range(3)]
        + [pltpu.VMEM((N,), f32) for _ in range(6)]
        + [pltpu.VMEM((ROWS_W,), f32) for _ in range(3)]
        + [pltpu.VMEM((CH * NPROW,), f32),
           pltpu.VMEM((64,), jnp.int32),
           pltpu.VMEM((2 * NWORD,), f32),
           pltpu.SemaphoreType.DMA((2,))]
    )
    fn = pl.kernel(_sc_select_body, out_type=out_type, mesh=mesh,
                   scratch_types=scratch,
                   compiler_params=pltpu.CompilerParams(
                       needs_layout_passes=False))
    return fn(xyz.reshape(-1), points.reshape(-1), nxyz_flat, hw.reshape(-1))


def _mlp_att_body(np_ref, fp_ref, wt_refs, b_refs, sc_refs, be_refs,
                  attp_ref, atth_ref, out_ref, *, rows):
    h = np_ref[0]
    g = fp_ref[0]
    for i in range(3):
        wt = wt_refs[i][...]
        bb = b_refs[i][...]
        sc = sc_refs[i][...]
        be = be_refs[i][...]
        h = jax.nn.relu((jnp.dot(h, wt, preferred_element_type=jnp.float32)
                         + bb) * sc + be)
        g = jax.nn.relu((jnp.dot(g, wt, preferred_element_type=jnp.float32)
                         + bb) * sc + be)
    delta_p = -np_ref[0][:, 0:3]
    e_p = jnp.dot(delta_p, attp_ref[...], preferred_element_type=jnp.float32)
    e_h = jnp.dot(h, atth_ref[...], preferred_element_type=jnp.float32)
    e_g = jnp.dot(g, atth_ref[...], preferred_element_type=jnp.float32)
    e = (e_p - e_h).reshape(rows, NSAMPLE, MLP_DIMS[2])
    e = e + e_g.reshape(rows, 1, MLP_DIMS[2])
    e = jnp.where(e >= 0, e, ALPHA * e)
    m = jnp.max(e, axis=1, keepdims=True)
    p = jnp.exp(e - m)
    att = p / jnp.sum(p, axis=1, keepdims=True)
    hn = h.reshape(rows, NSAMPLE, MLP_DIMS[2])
    out_ref[0] = jnp.sum(att * hn, axis=1)


def _run_mlp_att(np_feat, fp_feat, params):
    rows = 128
    grid = (B, NPOINT // rows)
    wts = [jnp.transpose(params['w%d' % i]) for i in range(3)]
    bs = [params['b%d' % i].reshape(1, -1) for i in range(3)]
    scs = [(params['g%d' % i] / jnp.sqrt(1.0 + EPS)).reshape(1, -1)
           for i in range(3)]
    bes = [params['be%d' % i].reshape(1, -1) for i in range(3)]
    att_p = params['att_a'][0:3]
    att_h = params['att_a'][3:]

    def fixed(shape):
        return pl.BlockSpec(shape, lambda b, r: (0,) * len(shape))

    in_specs = (
        [pl.BlockSpec((1, rows * NSAMPLE, 9), lambda b, r: (b, r, 0)),
         pl.BlockSpec((1, rows, 9), lambda b, r: (b, r, 0))]
        + [fixed(w.shape) for w in wts]
        + [fixed(x.shape) for x in bs]
        + [fixed(x.shape) for x in scs]
        + [fixed(x.shape) for x in bes]
        + [fixed(att_p.shape), fixed(att_h.shape)]
    )

    def body(np_r, fp_r, w0r, w1r, w2r, b0r, b1r, b2r, s0r, s1r, s2r,
             e0r, e1r, e2r, apr, ahr, out_r):
        _mlp_att_body(np_r, fp_r, (w0r, w1r, w2r), (b0r, b1r, b2r),
                      (s0r, s1r, s2r), (e0r, e1r, e2r), apr, ahr, out_r,
                      rows=rows)

    return pl.pallas_call(
        body,
        grid=grid,
        in_specs=in_specs,
        out_specs=pl.BlockSpec((1, rows, MLP_DIMS[2]), lambda b, r: (b, r, 0)),
        out_shape=jax.ShapeDtypeStruct((B, NPOINT, MLP_DIMS[2]), jnp.float32),
    )(np_feat, fp_feat, *wts, *bs, *scs, *bes, att_p, att_h)


def kernel(xyz, points, targets, w0, b0, g0, be0, w1, b1, g1, be1,
           w2, b2, g2, be2, att_a):
    params = {'w0': w0, 'b0': b0, 'g0': g0, 'be0': be0,
              'w1': w1, 'b1': b1, 'g1': g1, 'be1': be1,
              'w2': w2, 'b2': b2, 'g2': g2, 'be2': be2, 'att_a': att_a}
    start = jax.random.randint(jax.random.key(1), (B,), 0, N, dtype=jnp.int32)

    fps_idx = _run_fps(xyz, start)
    nxyz_flat, ntgt_flat, fp_flat = _run_sc_gather(
        xyz, points, targets, fps_idx)
    hw = _run_sqmask(fp_flat, xyz)
    np_flat = _run_sc_select(xyz, points, nxyz_flat, hw)

    new_xyz = nxyz_flat.reshape(B, 3, NPOINT)
    new_targets = ntgt_flat.reshape(B, NPOINT)
    np_feat = np_flat.reshape(B, NPOINT * NSAMPLE, 9)
    fp_feat = fp_flat.reshape(B, NPOINT, 9)

    pooled = _run_mlp_att(np_feat, fp_feat, params)

    return (new_xyz, jnp.swapaxes(pooled, 1, 2), new_targets)


# speedup vs baseline: 20.2647x; 1.1890x over previous
import numpy as np

import jax
import jax.numpy as jnp
from jax import lax
from jax.experimental import pallas as pl
from jax.experimental.pallas import tpu as pltpu
from jax.experimental.pallas import tpu_sc as plsc

B, N, D_FEAT = 16, 4096, 6
NPOINT, RADIUS, NSAMPLE = 1024, 0.2, 32
MLP_DIMS = [32, 32, 64]
EPS = 1e-5
ALPHA = 0.2
R2 = np.float32(RADIUS ** 2)

ROWS_W = NPOINT // 2
CH = 128
NPROW = NSAMPLE * 9
NWORD = N // 16


def _fps_body(xs_ref, ys_ref, zs_ref, start_ref, out_ref):
    xs = xs_ref[...]
    ys = ys_ref[...]
    zs = zs_ref[...]
    col_iota = lax.broadcasted_iota(jnp.int32, (B, N), 1)
    lane_iota = lax.broadcasted_iota(jnp.int32, (B, NPOINT), 1)

    def step(t, carry):
        dist, far, out = carry
        out = jnp.where(lane_iota == t, far, out)
        oh = (col_iota == far).astype(jnp.float32)
        cx = jnp.sum(xs * oh, axis=1, keepdims=True)
        cy = jnp.sum(ys * oh, axis=1, keepdims=True)
        cz = jnp.sum(zs * oh, axis=1, keepdims=True)
        d = (xs - cx) ** 2 + (ys - cy) ** 2
        d = d + (zs - cz) ** 2
        dist = jnp.minimum(dist, d)
        m = jnp.max(dist, axis=1, keepdims=True)
        far = jnp.min(jnp.where(dist == m, col_iota, N), axis=1, keepdims=True)
        return dist, far, out

    dist0 = jnp.full((B, N), 1e10, jnp.float32)
    out0 = jnp.zeros((B, NPOINT), jnp.int32)
    _, _, out = lax.fori_loop(0, NPOINT, step, (dist0, start_ref[...], out0))
    out_ref[...] = out


def _run_fps(xyz, start):
    return pl.pallas_call(
        _fps_body,
        out_shape=jax.ShapeDtypeStruct((B, NPOINT), jnp.int32),
    )(xyz[:, 0, :], xyz[:, 1, :], xyz[:, 2, :], start.reshape(B, 1))


def _sc_gather_body(xyz_hbm, pts_hbm, tgt_hbm, fidx_hbm,
                    nxyz_out, ntgt_out, fp_out,
                    xv, yv, zv, p0, p1, p2, p3, p4, p5,
                    tg, fidx, ntst, fpst, nxst):
    wid = lax.axis_index("s") * 2 + lax.axis_index("c")
    b = wid // 2
    h = wid % 2
    io = lax.iota(jnp.int32, 16)
    pv = (p0, p1, p2, p3, p4, p5)

    pltpu.sync_copy(xyz_hbm.at[pl.ds((b * 3 + 0) * N, N)], xv)
    pltpu.sync_copy(xyz_hbm.at[pl.ds((b * 3 + 1) * N, N)], yv)
    pltpu.sync_copy(xyz_hbm.at[pl.ds((b * 3 + 2) * N, N)], zv)
    for c in range(6):
        pltpu.sync_copy(pts_hbm.at[pl.ds((b * 6 + c) * N, N)], pv[c])
    pltpu.sync_copy(tgt_hbm.at[pl.ds(b * N, N)], tg)
    pltpu.sync_copy(fidx_hbm.at[pl.ds(b * NPOINT + h * ROWS_W, ROWS_W)], fidx)

    def fp_body(v, _):
        idx = fidx[pl.ds(v * 16, 16)]
        gx = plsc.load_gather(xv, [idx])
        gy = plsc.load_gather(yv, [idx])
        gz = plsc.load_gather(zv, [idx])
        nxst[pl.ds(v * 16, 16)] = gx
        nxst[pl.ds(ROWS_W + v * 16, 16)] = gy
        nxst[pl.ds(2 * ROWS_W + v * 16, 16)] = gz
        ntst[pl.ds(v * 16, 16)] = plsc.load_gather(tg, [idx])
        rbase = (io + v * 16) * 9
        plsc.store_scatter(fpst, [rbase], gx)
        plsc.store_scatter(fpst, [rbase + 1], gy)
        plsc.store_scatter(fpst, [rbase + 2], gz)
        for c in range(6):
            plsc.store_scatter(fpst, [rbase + 3 + c],
                               plsc.load_gather(pv[c], [idx]))
        return 0

    lax.fori_loop(0, ROWS_W // 16, fp_body, 0)

    for c in range(3):
        pltpu.sync_copy(
            nxst.at[pl.ds(c * ROWS_W, ROWS_W)],
            nxyz_out.at[pl.ds((b * 3 + c) * NPOINT + h * ROWS_W, ROWS_W)])
    pltpu.sync_copy(ntst, ntgt_out.at[pl.ds(b * NPOINT + h * ROWS_W, ROWS_W)])
    pltpu.sync_copy(fpst, fp_out.at[pl.ds((b * NPOINT + h * ROWS_W) * 9,
                                          ROWS_W * 9)])


def _run_sc_gather(xyz, points, targets, fps_idx):
    mesh = plsc.VectorSubcoreMesh(core_axis_name="c", subcore_axis_name="s")
    f32, i32 = jnp.float32, jnp.int32
    out_type = (
        jax.ShapeDtypeStruct((B * 3 * NPOINT,), f32),
        jax.ShapeDtypeStruct((B * NPOINT,), i32),
        jax.ShapeDtypeStruct((B * NPOINT * 9,), f32),
    )
    scratch = (
        [pltpu.VMEM((N,), f32) for _ in range(3)]
        + [pltpu.VMEM((N,), f32) for _ in range(6)]
        + [pltpu.VMEM((N,), i32),
           pltpu.VMEM((ROWS_W,), i32),
           pltpu.VMEM((ROWS_W,), i32),
           pltpu.VMEM((ROWS_W * 9,), f32),
           pltpu.VMEM((3 * ROWS_W,), f32)]
    )
    fn = pl.kernel(_sc_gather_body, out_type=out_type, mesh=mesh,
                   scratch_types=scratch,
                   compiler_params=pltpu.CompilerParams(
                       needs_layout_passes=False))
    return fn(xyz.reshape(-1), points.reshape(-1), targets.reshape(-1),
              fps_idx.reshape(-1))


def _sqmask_body(fp_ref, xyz_ref, pw_ref, out_ref):
    src = fp_ref[0][:, 0:3]
    xr = xyz_ref[0]
    mm = lax.dot_general(src, xr, (((1,), (0,)), ((), ())),
                         preferred_element_type=jnp.float32)
    s2 = src * src
    sn = (s2[:, 0:1] + s2[:, 1:2]) + s2[:, 2:3]
    x2 = xr * xr
    dn = (x2[0:1, :] + x2[1:2, :]) + x2[2:3, :]
    sq = -2.0 * mm
    sq = sq + sn
    sq = sq + dn
    inr = jnp.logical_not(sq > R2).astype(jnp.float32)
    out_ref[0] = jnp.dot(inr, pw_ref[...],
                         preferred_element_type=jnp.float32)


def _run_sqmask(fp_flat, xyz):
    ts = 256
    grid = (B, NPOINT // ts)
    fp_feat = fp_flat.reshape(B, NPOINT, 9)
    jr = np.arange(N)
    pw = np.where((jr[:, None] // 16) == np.arange(NWORD)[None, :],
                  (1 << (jr[:, None] % 16)).astype(np.float32), 0.0)
    pw = jnp.asarray(pw, jnp.float32)
    return pl.pallas_call(
        _sqmask_body,
        grid=grid,
        in_specs=[pl.BlockSpec((1, ts, 9), lambda b, r: (b, r, 0)),
                  pl.BlockSpec((1, 3, N), lambda b, r: (b, 0, 0)),
                  pl.BlockSpec((N, NWORD), lambda b, r: (0, 0))],
        out_specs=pl.BlockSpec((1, ts, NWORD), lambda b, r: (b, r, 0)),
        out_shape=jax.ShapeDtypeStruct((B, NPOINT, NWORD), jnp.float32),
    )(fp_feat, xyz, pw)


def _sc_select_body(xyz_hbm, pts_hbm, nxyz_hbm, hw_hbm, np_out,
                    xv, yv, zv, p0, p1, p2, p3, p4, p5,
                    nx, ny, nz, npst, selbuf, hwbuf, sems):
    wid = lax.axis_index("s") * 2 + lax.axis_index("c")
    b = wid // 2
    h = wid % 2
    io = lax.iota(jnp.int32, 16)
    pv = (p0, p1, p2, p3, p4, p5)
    rowbase = b * NPOINT + h * ROWS_W

    pltpu.sync_copy(xyz_hbm.at[pl.ds((b * 3 + 0) * N, N)], xv)
    pltpu.sync_copy(xyz_hbm.at[pl.ds((b * 3 + 1) * N, N)], yv)
    pltpu.sync_copy(xyz_hbm.at[pl.ds((b * 3 + 2) * N, N)], zv)
    for c in range(6):
        pltpu.sync_copy(pts_hbm.at[pl.ds((b * 6 + c) * N, N)], pv[c])
    pltpu.sync_copy(nxyz_hbm.at[pl.ds((b * 3 + 0) * NPOINT + h * ROWS_W,
                                      ROWS_W)], nx)
    pltpu.sync_copy(nxyz_hbm.at[pl.ds((b * 3 + 1) * NPOINT + h * ROWS_W,
                                      ROWS_W)], ny)
    pltpu.sync_copy(nxyz_hbm.at[pl.ds((b * 3 + 2) * NPOINT + h * ROWS_W,
                                      ROWS_W)], nz)

    pltpu.async_copy(hw_hbm.at[pl.ds(rowbase * NWORD, NWORD)],
                     hwbuf.at[pl.ds(0, NWORD)], sems.at[0])

    def row_body(r, _):
        par = lax.rem(r, 2)
        nxt = lax.rem(r + 1, 2)

        @pl.when(r + 1 < ROWS_W)
        def _prefetch():
            pltpu.async_copy(
                hw_hbm.at[pl.ds((rowbase + r + 1) * NWORD, NWORD)],
                hwbuf.at[pl.ds(nxt * NWORD, NWORD)], sems.at[nxt])

        pltpu.make_async_copy(
            hw_hbm.at[pl.ds((rowbase + r) * NWORD, NWORD)],
            hwbuf.at[pl.ds(par * NWORD, NWORD)], sems.at[par]).wait()

        hwoff = par * NWORD

        def sel_cond(c):
            pos_v, j = c
            return jnp.logical_and(jnp.max(pos_v) < NSAMPLE, j < NWORD // 4)

        def sel_body(c):
            pos_v, j = c
            for k in range(4):
                wf = plsc.load_gather(
                    hwbuf,
                    [jnp.broadcast_to(hwoff + j * 4 + k,
                                      (16,)).astype(jnp.int32)])
                w = wf.astype(jnp.int32)
                m = ((w >> io) & 1) == 1
                rank = plsc.cumsum(m.astype(jnp.int32))
                plsc.store_scatter(selbuf, [pos_v + (rank - 1)],
                                   io + (j * 4 + k) * 16, mask=m)
                pos_v = pos_v + plsc.all_reduce_population_count(m)
            return pos_v, j + 1

        pos_v, _ = lax.while_loop(sel_cond, sel_body,
                                  (jnp.zeros((16,), jnp.int32), jnp.int32(0)))
        pos = jnp.max(pos_v)
        s0 = selbuf[pl.ds(0, 16)]
        s1 = selbuf[pl.ds(16, 16)]
        first = jnp.broadcast_to(s0[0], (16,))
        idx0 = jnp.where(io < pos, s0, first)
        idx1 = jnp.where(io + 16 < pos, s1, first)

        rsplat = jnp.broadcast_to(r, (16,)).astype(jnp.int32)
        cx = plsc.load_gather(nx, [rsplat])
        cy = plsc.load_gather(ny, [rsplat])
        cz = plsc.load_gather(nz, [rsplat])
        ro = lax.rem(r, CH) * NPROW
        for half, idxh in ((0, idx0), (1, idx1)):
            sb = ro + (io + half * 16) * 9
            plsc.store_scatter(npst, [sb], plsc.load_gather(xv, [idxh]) - cx)
            plsc.store_scatter(npst, [sb + 1],
                               plsc.load_gather(yv, [idxh]) - cy)
            plsc.store_scatter(npst, [sb + 2],
                               plsc.load_gather(zv, [idxh]) - cz)
            for c in range(6):
                plsc.store_scatter(npst, [sb + 3 + c],
                                   plsc.load_gather(pv[c], [idxh]))

        @pl.when(lax.rem(r, CH) == CH - 1)
        def _flush():
            off = (rowbase + (r - (CH - 1))) * NPROW
            pltpu.sync_copy(npst, np_out.at[pl.ds(off, CH * NPROW)])

        return 0

    lax.fori_loop(0, ROWS_W, row_body, 0)


def _run_sc_select(xyz, points, nxyz_flat, hw):
    mesh = plsc.VectorSubcoreMesh(core_axis_name="c", subcore_axis_name="s")
    f32 = jnp.float32
    out_type = jax.ShapeDtypeStruct((B * NPOINT * NPROW,), f32)
    scratch = (
        [pltpu.VMEM((N,), f32) for _ in range(3)]
        + [pltpu.VMEM((N,), f32) for _ in range(6)]
        + [pltpu.VMEM((ROWS_W,), f32) for _ in range(3)]
        + [pltpu.VMEM((CH * NPROW,), f32),
           pltpu.VMEM((128,), jnp.int32),
           pltpu.VMEM((2 * NWORD,), f32),
           pltpu.SemaphoreType.DMA((2,))]
    )
    fn = pl.kernel(_sc_select_body, out_type=out_type, mesh=mesh,
                   scratch_types=scratch,
                   compiler_params=pltpu.CompilerParams(
                       needs_layout_passes=False))
    return fn(xyz.reshape(-1), points.reshape(-1), nxyz_flat, hw.reshape(-1))


def _mlp_att_body(np_ref, fp_ref, wt_refs, b_refs, sc_refs, be_refs,
                  attp_ref, atth_ref, out_ref, *, rows):
    h = np_ref[0]
    g = fp_ref[0]
    for i in range(3):
        wt = wt_refs[i][...]
        bb = b_refs[i][...]
        sc = sc_refs[i][...]
        be = be_refs[i][...]
        h = jax.nn.relu((jnp.dot(h, wt, preferred_element_type=jnp.float32)
                         + bb) * sc + be)
        g = jax.nn.relu((jnp.dot(g, wt, preferred_element_type=jnp.float32)
                         + bb) * sc + be)
    delta_p = -np_ref[0][:, 0:3]
    e_p = jnp.dot(delta_p, attp_ref[...], preferred_element_type=jnp.float32)
    e_h = jnp.dot(h, atth_ref[...], preferred_element_type=jnp.float32)
    e_g = jnp.dot(g, atth_ref[...], preferred_element_type=jnp.float32)
    e = (e_p - e_h).reshape(rows, NSAMPLE, MLP_DIMS[2])
    e = e + e_g.reshape(rows, 1, MLP_DIMS[2])
    e = jnp.where(e >= 0, e, ALPHA * e)
    m = jnp.max(e, axis=1, keepdims=True)
    p = jnp.exp(e - m)
    att = p / jnp.sum(p, axis=1, keepdims=True)
    hn = h.reshape(rows, NSAMPLE, MLP_DIMS[2])
    out_ref[0] = jnp.sum(att * hn, axis=1)


def _run_mlp_att(np_feat, fp_feat, params):
    rows = 128
    grid = (B, NPOINT // rows)
    wts = [jnp.transpose(params['w%d' % i]) for i in range(3)]
    bs = [params['b%d' % i].reshape(1, -1) for i in range(3)]
    scs = [(params['g%d' % i] / jnp.sqrt(1.0 + EPS)).reshape(1, -1)
           for i in range(3)]
    bes = [params['be%d' % i].reshape(1, -1) for i in range(3)]
    att_p = params['att_a'][0:3]
    att_h = params['att_a'][3:]

    def fixed(shape):
        return pl.BlockSpec(shape, lambda b, r: (0,) * len(shape))

    in_specs = (
        [pl.BlockSpec((1, rows * NSAMPLE, 9), lambda b, r: (b, r, 0)),
         pl.BlockSpec((1, rows, 9), lambda b, r: (b, r, 0))]
        + [fixed(w.shape) for w in wts]
        + [fixed(x.shape) for x in bs]
        + [fixed(x.shape) for x in scs]
        + [fixed(x.shape) for x in bes]
        + [fixed(att_p.shape), fixed(att_h.shape)]
    )

    def body(np_r, fp_r, w0r, w1r, w2r, b0r, b1r, b2r, s0r, s1r, s2r,
             e0r, e1r, e2r, apr, ahr, out_r):
        _mlp_att_body(np_r, fp_r, (w0r, w1r, w2r), (b0r, b1r, b2r),
                      (s0r, s1r, s2r), (e0r, e1r, e2r), apr, ahr, out_r,
                      rows=rows)

    return pl.pallas_call(
        body,
        grid=grid,
        in_specs=in_specs,
        out_specs=pl.BlockSpec((1, rows, MLP_DIMS[2]), lambda b, r: (b, r, 0)),
        out_shape=jax.ShapeDtypeStruct((B, NPOINT, MLP_DIMS[2]), jnp.float32),
    )(np_feat, fp_feat, *wts, *bs, *scs, *bes, att_p, att_h)


def kernel(xyz, points, targets, w0, b0, g0, be0, w1, b1, g1, be1,
           w2, b2, g2, be2, att_a):
    params = {'w0': w0, 'b0': b0, 'g0': g0, 'be0': be0,
              'w1': w1, 'b1': b1, 'g1': g1, 'be1': be1,
              'w2': w2, 'b2': b2, 'g2': g2, 'be2': be2, 'att_a': att_a}
    start = jax.random.randint(jax.random.key(1), (B,), 0, N, dtype=jnp.int32)

    fps_idx = _run_fps(xyz, start)
    nxyz_flat, ntgt_flat, fp_flat = _run_sc_gather(
        xyz, points, targets, fps_idx)
    hw = _run_sqmask(fp_flat, xyz)
    np_flat = _run_sc_select(xyz, points, nxyz_flat, hw)

    new_xyz = nxyz_flat.reshape(B, 3, NPOINT)
    new_targets = ntgt_flat.reshape(B, NPOINT)
    np_feat = np_flat.reshape(B, NPOINT * NSAMPLE, 9)
    fp_feat = fp_flat.reshape(B, NPOINT, 9)

    pooled = _run_mlp_att(np_feat, fp_feat, params)

    return (new_xyz, jnp.swapaxes(pooled, 1, 2), new_targets)


# speedup vs baseline: 28.0531x; 1.3843x over previous
import numpy as np

import jax
import jax.numpy as jnp
from jax import lax
from jax.experimental import pallas as pl
from jax.experimental.pallas import tpu as pltpu
from jax.experimental.pallas import tpu_sc as plsc

B, N, D_FEAT = 16, 4096, 6
NPOINT, RADIUS, NSAMPLE = 1024, 0.2, 32
MLP_DIMS = [32, 32, 64]
EPS = 1e-5
ALPHA = 0.2
R2 = np.float32(RADIUS ** 2)

ROWS_W = NPOINT // 2
CH = 128
NPROW = NSAMPLE * 9
NWORD = N // 16


def _fps_body(xs_ref, ys_ref, zs_ref, start_ref, out_ref):
    xs = xs_ref[...]
    ys = ys_ref[...]
    zs = zs_ref[...]
    col_iota = lax.broadcasted_iota(jnp.int32, (B, N), 1)
    lane_iota = lax.broadcasted_iota(jnp.int32, (B, NPOINT), 1)

    def step(t, carry):
        dist, far, out = carry
        out = jnp.where(lane_iota == t, far, out)
        oh = (col_iota == far).astype(jnp.float32)
        cx = jnp.sum(xs * oh, axis=1, keepdims=True)
        cy = jnp.sum(ys * oh, axis=1, keepdims=True)
        cz = jnp.sum(zs * oh, axis=1, keepdims=True)
        d = (xs - cx) ** 2 + (ys - cy) ** 2
        d = d + (zs - cz) ** 2
        dist = jnp.minimum(dist, d)
        m = jnp.max(dist, axis=1, keepdims=True)
        far = jnp.min(jnp.where(dist == m, col_iota, N), axis=1, keepdims=True)
        return dist, far, out

    dist0 = jnp.full((B, N), 1e10, jnp.float32)
    out0 = jnp.zeros((B, NPOINT), jnp.int32)
    _, _, out = lax.fori_loop(0, NPOINT, step, (dist0, start_ref[...], out0))
    out_ref[...] = out


def _run_fps(xyz, start):
    return pl.pallas_call(
        _fps_body,
        out_shape=jax.ShapeDtypeStruct((B, NPOINT), jnp.int32),
    )(xyz[:, 0, :], xyz[:, 1, :], xyz[:, 2, :], start.reshape(B, 1))


def _sc_gather_body(xyz_hbm, pts_hbm, tgt_hbm, fidx_hbm,
                    nxyz_out, ntgt_out, fp_out,
                    xv, yv, zv, p0, p1, p2, p3, p4, p5,
                    tg, fidx, ntst, fpst, nxst):
    wid = lax.axis_index("s") * 2 + lax.axis_index("c")
    b = wid // 2
    h = wid % 2
    io = lax.iota(jnp.int32, 16)
    pv = (p0, p1, p2, p3, p4, p5)

    pltpu.sync_copy(xyz_hbm.at[pl.ds((b * 3 + 0) * N, N)], xv)
    pltpu.sync_copy(xyz_hbm.at[pl.ds((b * 3 + 1) * N, N)], yv)
    pltpu.sync_copy(xyz_hbm.at[pl.ds((b * 3 + 2) * N, N)], zv)
    for c in range(6):
        pltpu.sync_copy(pts_hbm.at[pl.ds((b * 6 + c) * N, N)], pv[c])
    pltpu.sync_copy(tgt_hbm.at[pl.ds(b * N, N)], tg)
    pltpu.sync_copy(fidx_hbm.at[pl.ds(b * NPOINT + h * ROWS_W, ROWS_W)], fidx)

    def fp_body(v, _):
        idx = fidx[pl.ds(v * 16, 16)]
        gx = plsc.load_gather(xv, [idx])
        gy = plsc.load_gather(yv, [idx])
        gz = plsc.load_gather(zv, [idx])
        nxst[pl.ds(v * 16, 16)] = gx
        nxst[pl.ds(ROWS_W + v * 16, 16)] = gy
        nxst[pl.ds(2 * ROWS_W + v * 16, 16)] = gz
        ntst[pl.ds(v * 16, 16)] = plsc.load_gather(tg, [idx])
        rbase = (io + v * 16) * 9
        plsc.store_scatter(fpst, [rbase], gx)
        plsc.store_scatter(fpst, [rbase + 1], gy)
        plsc.store_scatter(fpst, [rbase + 2], gz)
        for c in range(6):
            plsc.store_scatter(fpst, [rbase + 3 + c],
                               plsc.load_gather(pv[c], [idx]))
        return 0

    lax.fori_loop(0, ROWS_W // 16, fp_body, 0)

    for c in range(3):
        pltpu.sync_copy(
            nxst.at[pl.ds(c * ROWS_W, ROWS_W)],
            nxyz_out.at[pl.ds((b * 3 + c) * NPOINT + h * ROWS_W, ROWS_W)])
    pltpu.sync_copy(ntst, ntgt_out.at[pl.ds(b * NPOINT + h * ROWS_W, ROWS_W)])
    pltpu.sync_copy(fpst, fp_out.at[pl.ds((b * NPOINT + h * ROWS_W) * 9,
                                          ROWS_W * 9)])


def _run_sc_gather(xyz, points, targets, fps_idx):
    mesh = plsc.VectorSubcoreMesh(core_axis_name="c", subcore_axis_name="s")
    f32, i32 = jnp.float32, jnp.int32
    out_type = (
        jax.ShapeDtypeStruct((B * 3 * NPOINT,), f32),
        jax.ShapeDtypeStruct((B * NPOINT,), i32),
        jax.ShapeDtypeStruct((B * NPOINT * 9,), f32),
    )
    scratch = (
        [pltpu.VMEM((N,), f32) for _ in range(3)]
        + [pltpu.VMEM((N,), f32) for _ in range(6)]
        + [pltpu.VMEM((N,), i32),
           pltpu.VMEM((ROWS_W,), i32),
           pltpu.VMEM((ROWS_W,), i32),
           pltpu.VMEM((ROWS_W * 9,), f32),
           pltpu.VMEM((3 * ROWS_W,), f32)]
    )
    fn = pl.kernel(_sc_gather_body, out_type=out_type, mesh=mesh,
                   scratch_types=scratch,
                   compiler_params=pltpu.CompilerParams(
                       needs_layout_passes=False))
    return fn(xyz.reshape(-1), points.reshape(-1), targets.reshape(-1),
              fps_idx.reshape(-1))


def _sqmask_body(fp_ref, xyz_ref, pw_ref, out_ref):
    src = fp_ref[0][:, 0:3]
    xr = xyz_ref[0]
    mm = lax.dot_general(src, xr, (((1,), (0,)), ((), ())),
                         preferred_element_type=jnp.float32)
    s2 = src * src
    sn = (s2[:, 0:1] + s2[:, 1:2]) + s2[:, 2:3]
    x2 = xr * xr
    dn = (x2[0:1, :] + x2[1:2, :]) + x2[2:3, :]
    sq = -2.0 * mm
    sq = sq + sn
    sq = sq + dn
    inr = jnp.logical_not(sq > R2).astype(jnp.float32)
    out_ref[0] = jnp.dot(inr, pw_ref[...],
                         preferred_element_type=jnp.float32)


def _run_sqmask(fp_flat, xyz):
    ts = 256
    grid = (B, NPOINT // ts)
    fp_feat = fp_flat.reshape(B, NPOINT, 9)
    jr = np.arange(N)
    pw = np.where((jr[:, None] // 16) == np.arange(NWORD)[None, :],
                  (1 << (jr[:, None] % 16)).astype(np.float32), 0.0)
    pw = jnp.asarray(pw, jnp.float32)
    return pl.pallas_call(
        _sqmask_body,
        grid=grid,
        in_specs=[pl.BlockSpec((1, ts, 9), lambda b, r: (b, r, 0)),
                  pl.BlockSpec((1, 3, N), lambda b, r: (b, 0, 0)),
                  pl.BlockSpec((N, NWORD), lambda b, r: (0, 0))],
        out_specs=pl.BlockSpec((1, ts, NWORD), lambda b, r: (b, r, 0)),
        out_shape=jax.ShapeDtypeStruct((B, NPOINT, NWORD), jnp.float32),
    )(fp_feat, xyz, pw)


def _sc_select_body(xyz_hbm, pts_hbm, nxyz_hbm, hw_hbm, np_out,
                    xv, yv, zv, p0, p1, p2, p3, p4, p5,
                    nx, ny, nz, npst, selbuf, hwbuf, sems):
    wid = lax.axis_index("s") * 2 + lax.axis_index("c")
    b = wid // 2
    h = wid % 2
    io = lax.iota(jnp.int32, 16)
    pv = (p0, p1, p2, p3, p4, p5)
    rowbase = b * NPOINT + h * ROWS_W

    pltpu.sync_copy(xyz_hbm.at[pl.ds((b * 3 + 0) * N, N)], xv)
    pltpu.sync_copy(xyz_hbm.at[pl.ds((b * 3 + 1) * N, N)], yv)
    pltpu.sync_copy(xyz_hbm.at[pl.ds((b * 3 + 2) * N, N)], zv)
    for c in range(6):
        pltpu.sync_copy(pts_hbm.at[pl.ds((b * 6 + c) * N, N)], pv[c])
    pltpu.sync_copy(nxyz_hbm.at[pl.ds((b * 3 + 0) * NPOINT + h * ROWS_W,
                                      ROWS_W)], nx)
    pltpu.sync_copy(nxyz_hbm.at[pl.ds((b * 3 + 1) * NPOINT + h * ROWS_W,
                                      ROWS_W)], ny)
    pltpu.sync_copy(nxyz_hbm.at[pl.ds((b * 3 + 2) * NPOINT + h * ROWS_W,
                                      ROWS_W)], nz)

    RC = 8
    pltpu.async_copy(hw_hbm.at[pl.ds(rowbase * NWORD, RC * NWORD)],
                     hwbuf.at[pl.ds(0, RC * NWORD)], sems.at[0])

    def chunk_body(ck, _):
        par = lax.rem(ck, 2)
        nxt = lax.rem(ck + 1, 2)

        @pl.when(ck + 1 < ROWS_W // RC)
        def _prefetch():
            pltpu.async_copy(
                hw_hbm.at[pl.ds((rowbase + (ck + 1) * RC) * NWORD,
                                RC * NWORD)],
                hwbuf.at[pl.ds(nxt * RC * NWORD, RC * NWORD)], sems.at[nxt])

        pltpu.make_async_copy(
            hw_hbm.at[pl.ds((rowbase + ck * RC) * NWORD, RC * NWORD)],
            hwbuf.at[pl.ds(par * RC * NWORD, RC * NWORD)], sems.at[par]).wait()

        def row_body(ri, _):
            r = ck * RC + ri
            hwoff = par * RC * NWORD + ri * NWORD

            def sel_cond(c):
                pos_v, j = c
                return jnp.logical_and(jnp.max(pos_v) < NSAMPLE,
                                       j < NWORD // 8)

            def sel_body(c):
                pos_v, j = c
                ms = []
                pcs = []
                for k in range(8):
                    wf = plsc.load_gather(
                        hwbuf,
                        [jnp.broadcast_to(hwoff + j * 8 + k,
                                          (16,)).astype(jnp.int32)])
                    w = wf.astype(jnp.int32)
                    m = ((w >> io) & 1) == 1
                    ms.append(m)
                    pcs.append(plsc.all_reduce_population_count(m))
                off = pos_v
                for k in range(8):
                    rank = plsc.cumsum(ms[k].astype(jnp.int32))
                    plsc.store_scatter(selbuf, [off + (rank - 1)],
                                       io + (j * 8 + k) * 16, mask=ms[k])
                    off = off + pcs[k]
                return off, j + 1

            pos_v, _ = lax.while_loop(
                sel_cond, sel_body,
                (jnp.zeros((16,), jnp.int32), jnp.int32(0)))
            pos = jnp.max(pos_v)
            s0 = selbuf[pl.ds(0, 16)]
            s1 = selbuf[pl.ds(16, 16)]
            first = jnp.broadcast_to(s0[0], (16,))
            idx0 = jnp.where(io < pos, s0, first)
            idx1 = jnp.where(io + 16 < pos, s1, first)

            rsplat = jnp.broadcast_to(r, (16,)).astype(jnp.int32)
            cx = plsc.load_gather(nx, [rsplat])
            cy = plsc.load_gather(ny, [rsplat])
            cz = plsc.load_gather(nz, [rsplat])
            ro = lax.rem(r, CH) * NPROW
            for half, idxh in ((0, idx0), (1, idx1)):
                sb = ro + (io + half * 16) * 9
                plsc.store_scatter(npst, [sb],
                                   plsc.load_gather(xv, [idxh]) - cx)
                plsc.store_scatter(npst, [sb + 1],
                                   plsc.load_gather(yv, [idxh]) - cy)
                plsc.store_scatter(npst, [sb + 2],
                                   plsc.load_gather(zv, [idxh]) - cz)
                for c in range(6):
                    plsc.store_scatter(npst, [sb + 3 + c],
                                       plsc.load_gather(pv[c], [idxh]))

            @pl.when(lax.rem(r, CH) == CH - 1)
            def _flush():
                off = (rowbase + (r - (CH - 1))) * NPROW
                pltpu.sync_copy(npst, np_out.at[pl.ds(off, CH * NPROW)])

            return 0

        lax.fori_loop(0, RC, row_body, 0)
        return 0

    lax.fori_loop(0, ROWS_W // RC, chunk_body, 0)


def _run_sc_select(xyz, points, nxyz_flat, hw):
    mesh = plsc.VectorSubcoreMesh(core_axis_name="c", subcore_axis_name="s")
    f32 = jnp.float32
    out_type = jax.ShapeDtypeStruct((B * NPOINT * NPROW,), f32)
    scratch = (
        [pltpu.VMEM((N,), f32) for _ in range(3)]
        + [pltpu.VMEM((N,), f32) for _ in range(6)]
        + [pltpu.VMEM((ROWS_W,), f32) for _ in range(3)]
        + [pltpu.VMEM((CH * NPROW,), f32),
           pltpu.VMEM((192,), jnp.int32),
           pltpu.VMEM((2 * 8 * NWORD,), f32),
           pltpu.SemaphoreType.DMA((2,))]
    )
    fn = pl.kernel(_sc_select_body, out_type=out_type, mesh=mesh,
                   scratch_types=scratch,
                   compiler_params=pltpu.CompilerParams(
                       needs_layout_passes=False))
    return fn(xyz.reshape(-1), points.reshape(-1), nxyz_flat, hw.reshape(-1))


def _mlp_att_body(np_ref, fp_ref, wt_refs, b_refs, sc_refs, be_refs,
                  attp_ref, atth_ref, out_ref, *, rows):
    h = np_ref[0]
    g = fp_ref[0]
    for i in range(3):
        wt = wt_refs[i][...]
        bb = b_refs[i][...]
        sc = sc_refs[i][...]
        be = be_refs[i][...]
        h = jax.nn.relu((jnp.dot(h, wt, preferred_element_type=jnp.float32)
                         + bb) * sc + be)
        g = jax.nn.relu((jnp.dot(g, wt, preferred_element_type=jnp.float32)
                         + bb) * sc + be)
    delta_p = -np_ref[0][:, 0:3]
    e_p = jnp.dot(delta_p, attp_ref[...], preferred_element_type=jnp.float32)
    e_h = jnp.dot(h, atth_ref[...], preferred_element_type=jnp.float32)
    e_g = jnp.dot(g, atth_ref[...], preferred_element_type=jnp.float32)
    e = (e_p - e_h).reshape(rows, NSAMPLE, MLP_DIMS[2])
    e = e + e_g.reshape(rows, 1, MLP_DIMS[2])
    e = jnp.where(e >= 0, e, ALPHA * e)
    m = jnp.max(e, axis=1, keepdims=True)
    p = jnp.exp(e - m)
    att = p / jnp.sum(p, axis=1, keepdims=True)
    hn = h.reshape(rows, NSAMPLE, MLP_DIMS[2])
    out_ref[0] = jnp.sum(att * hn, axis=1)


def _run_mlp_att(np_feat, fp_feat, params):
    rows = 128
    grid = (B, NPOINT // rows)
    wts = [jnp.transpose(params['w%d' % i]) for i in range(3)]
    bs = [params['b%d' % i].reshape(1, -1) for i in range(3)]
    scs = [(params['g%d' % i] / jnp.sqrt(1.0 + EPS)).reshape(1, -1)
           for i in range(3)]
    bes = [params['be%d' % i].reshape(1, -1) for i in range(3)]
    att_p = params['att_a'][0:3]
    att_h = params['att_a'][3:]

    def fixed(shape):
        return pl.BlockSpec(shape, lambda b, r: (0,) * len(shape))

    in_specs = (
        [pl.BlockSpec((1, rows * NSAMPLE, 9), lambda b, r: (b, r, 0)),
         pl.BlockSpec((1, rows, 9), lambda b, r: (b, r, 0))]
        + [fixed(w.shape) for w in wts]
        + [fixed(x.shape) for x in bs]
        + [fixed(x.shape) for x in scs]
        + [fixed(x.shape) for x in bes]
        + [fixed(att_p.shape), fixed(att_h.shape)]
    )

    def body(np_r, fp_r, w0r, w1r, w2r, b0r, b1r, b2r, s0r, s1r, s2r,
             e0r, e1r, e2r, apr, ahr, out_r):
        _mlp_att_body(np_r, fp_r, (w0r, w1r, w2r), (b0r, b1r, b2r),
                      (s0r, s1r, s2r), (e0r, e1r, e2r), apr, ahr, out_r,
                      rows=rows)

    return pl.pallas_call(
        body,
        grid=grid,
        in_specs=in_specs,
        out_specs=pl.BlockSpec((1, rows, MLP_DIMS[2]), lambda b, r: (b, r, 0)),
        out_shape=jax.ShapeDtypeStruct((B, NPOINT, MLP_DIMS[2]), jnp.float32),
    )(np_feat, fp_feat, *wts, *bs, *scs, *bes, att_p, att_h)


def kernel(xyz, points, targets, w0, b0, g0, be0, w1, b1, g1, be1,
           w2, b2, g2, be2, att_a):
    params = {'w0': w0, 'b0': b0, 'g0': g0, 'be0': be0,
              'w1': w1, 'b1': b1, 'g1': g1, 'be1': be1,
              'w2': w2, 'b2': b2, 'g2': g2, 'be2': be2, 'att_a': att_a}
    start = jax.random.randint(jax.random.key(1), (B,), 0, N, dtype=jnp.int32)

    fps_idx = _run_fps(xyz, start)
    nxyz_flat, ntgt_flat, fp_flat = _run_sc_gather(
        xyz, points, targets, fps_idx)
    hw = _run_sqmask(fp_flat, xyz)
    np_flat = _run_sc_select(xyz, points, nxyz_flat, hw)

    new_xyz = nxyz_flat.reshape(B, 3, NPOINT)
    new_targets = ntgt_flat.reshape(B, NPOINT)
    np_feat = np_flat.reshape(B, NPOINT * NSAMPLE, 9)
    fp_feat = fp_flat.reshape(B, NPOINT, 9)

    pooled = _run_mlp_att(np_feat, fp_feat, params)

    return (new_xyz, jnp.swapaxes(pooled, 1, 2), new_targets)
